# baseline (speedup 1.0000x reference)
import dataclasses
import functools

import jax
import jax.numpy as jnp
from jax import lax
from jax.experimental import pallas as pl
from jax.experimental.pallas import tpu as pltpu
from jax.experimental.pallas import tpu_sc as plsc

K = 1000000
H_GRID = 200
W_GRID = 70400

NW = 32
BSH = 9
GW = 512
NB = (W_GRID + GW - 1) // GW
NBP = 144
GRID_CELLS = H_GRID * GW
DUMP = GRID_CELLS
GRID_TOT = GRID_CELLS + 16
CAP = 432
NEG = -9999999.0

NVEC = K // 16
VPW = NVEC // NW
VEXTRA = NVEC - VPW * NW
CHUNK_V = 64
CHUNK_P = CHUNK_V * 16
NCHUNK = (VPW + 1 + CHUNK_V - 1) // CHUNK_V

GRP = 8
NGRP = NW // GRP

BPW = NB // NW
WEX = NB - BPW * NW


def _mlp_body(h_ref, w1_ref, b1_ref, w2_ref, b2_ref, w3_ref, b3_ref,
              w4_ref, b4_ref, o_ref):
    a = h_ref[...].reshape(4, h_ref.shape[3])
    z = jnp.maximum(jnp.dot(w1_ref[...], a,
                            preferred_element_type=jnp.float32) + b1_ref[...], 0.0)
    z = jnp.maximum(jnp.dot(w2_ref[...], z,
                            preferred_element_type=jnp.float32) + b2_ref[...], 0.0)
    z = jnp.maximum(jnp.dot(w3_ref[...], z,
                            preferred_element_type=jnp.float32) + b3_ref[...], 0.0)
    o_ref[...] = (jnp.dot(w4_ref[...], z,
                          preferred_element_type=jnp.float32)
                  + b4_ref[...]).reshape(o_ref.shape[0])


def _mlp(h2d, W1, b1, W2, b2, W3, b3, W4, b4):
    BK = 32768
    grid = (pl.cdiv(K, BK),)
    full = lambda shp: pl.BlockSpec(shp, lambda i: (0, 0))
    return pl.pallas_call(
        _mlp_body,
        grid=grid,
        in_specs=[
            pl.BlockSpec((1, 4, 1, BK), lambda i: (0, 0, 0, i)),
            full((18, 4)), full((18, 1)),
            full((36, 18)), full((36, 1)),
            full((36, 36)), full((36, 1)),
            full((1, 36)), full((1, 1)),
        ],
        out_specs=pl.BlockSpec((BK,), lambda i: (i,)),
        out_shape=jax.ShapeDtypeStruct((K,), jnp.float32),
    )(h2d, W1, b1[:, None], W2, b2[:, None], W3, b3[:, None], W4, b4[:, None])


def _route_kernel(h_hbm, w_hbm, x_hbm, cells_hbm, xs_hbm, counts_hbm,
                  stage_c, stage_x, counts,
                  hbuf0, wbuf0, xbuf0, hbuf1, wbuf1, xbuf1, sem0, sem1):
    wid = lax.axis_index("s") * 2 + lax.axis_index("c")
    v0 = wid * VPW + jnp.minimum(wid, VEXTRA)
    nv = VPW + jnp.where(wid < VEXTRA, 1, 0)

    zero16 = jnp.zeros((16,), jnp.int32)
    for j in range(0, NBP, 16):
        counts[pl.ds(j, 16)] = zero16

    iota = lax.iota(jnp.int32, 16)

    bufs = [(hbuf0, wbuf0, xbuf0), (hbuf1, wbuf1, xbuf1)]
    sems = [sem0, sem1]

    def issue(c, bi):
        off = (v0 + c * CHUNK_V) * 16
        offc = jnp.minimum(off, K - CHUNK_P)
        return [
            pltpu.async_copy(h_hbm.at[pl.ds(offc, CHUNK_P)],
                             bufs[bi][0], sems[bi]),
            pltpu.async_copy(w_hbm.at[pl.ds(offc, CHUNK_P)],
                             bufs[bi][1], sems[bi]),
            pltpu.async_copy(x_hbm.at[pl.ds(offc, CHUNK_P)],
                             bufs[bi][2], sems[bi]),
        ]

    hs = issue(0, 0)
    for c in range(NCHUNK):
        hs_next = issue(c + 1, (c + 1) % 2) if c + 1 < NCHUNK else None
        for hdl in hs:
            hdl.wait()
        hs = hs_next
        hb, wb, xb = bufs[c % 2]
        off = (v0 + c * CHUNK_V) * 16
        base = off - jnp.minimum(off, K - CHUNK_P)
        nproc = jnp.clip(nv - c * CHUNK_V, 0, CHUNK_V)

        def do_vec(v, hb, wb, xb, base):
            p = base + v * 16
            hh = hb[pl.ds(p, 16)]
            ww = wb[pl.ds(p, 16)]
            xx = xb[pl.ds(p, 16)]
            b = lax.shift_right_logical(ww, BSH)
            wl = ww & (GW - 1)
            cell = hh * GW + wl
            cnt, last = plsc.scan_count(b)
            basec = plsc.load_gather(counts, [b])
            slot = basec + cnt - 1
            addr = b * CAP + slot
            plsc.store_scatter(stage_c, [addr], cell)
            plsc.store_scatter(stage_x, [addr], xx)
            plsc.store_scatter(counts, [b], basec + cnt, mask=last)

        def body2(i, carry, hb=hb, wb=wb, xb=xb, base=base):
            do_vec(2 * i, hb, wb, xb, base)
            do_vec(2 * i + 1, hb, wb, xb, base)
            return carry

        lax.fori_loop(0, lax.shift_right_logical(nproc, 1), body2, 0)

        @pl.when((nproc & 1) == 1)
        def _(hb=hb, wb=wb, xb=xb, base=base, nproc=nproc):
            do_vec(nproc - 1, hb, wb, xb, base)

    dump16 = jnp.full((16,), DUMP, jnp.int32)
    neg16 = jnp.full((16,), NEG, jnp.float32)

    for g in range(NBP // 16):
        n16 = counts[pl.ds(g * 16, 16)]
        pad16 = (-n16) & 15
        bvec = g * 16 + iota
        base_addr = bvec * CAP + n16
        for j in range(15):
            mask = (j < pad16) & (bvec < NB)
            plsc.store_scatter(stage_c, [base_addr + j], dump16, mask=mask)
            plsc.store_scatter(stage_x, [base_addr + j], neg16, mask=mask)
        counts[pl.ds(g * 16, 16)] = n16 + pad16

    pltpu.sync_copy(stage_c, cells_hbm.at[pl.ds(wid * (NB * CAP), NB * CAP)])
    pltpu.sync_copy(stage_x, xs_hbm.at[pl.ds(wid * (NB * CAP), NB * CAP)])
    pltpu.sync_copy(counts, counts_hbm.at[pl.ds(wid * NBP, NBP)])


def _reduce_kernel(cells_hbm, xs_hbm, counts_hbm, out_hbm,
                   grid_v, cbufA, xbufA, cbufB, xbufB, cnts_all, col,
                   semA, semB):
    wid = lax.axis_index("s") * 2 + lax.axis_index("c")
    neg16 = jnp.full((16,), NEG, jnp.float32)
    b0 = wid * BPW + jnp.minimum(wid, WEX)
    nb = BPW + jnp.where(wid < WEX, 1, 0)
    ntask = nb * NGRP

    def ig(j, carry):
        for u in range(16):
            grid_v[pl.ds(j * 256 + u * 16, 16)] = neg16
        return carry

    lax.fori_loop(0, GRID_CELLS // 256, ig, 0)
    grid_v[pl.ds(GRID_CELLS, 16)] = neg16

    pltpu.sync_copy(counts_hbm, cnts_all)

    def issue_task(t, cb, xb, sem):
        b = b0 + lax.shift_right_logical(t, 2)
        g = t & 3

        def di(j, carry):
            w = g * GRP + j
            src = (w * NB + b) * CAP
            pltpu.async_copy(cells_hbm.at[pl.ds(src, CAP)],
                             cb.at[pl.ds(j * CAP, CAP)], sem)
            pltpu.async_copy(xs_hbm.at[pl.ds(src, CAP)],
                             xb.at[pl.ds(j * CAP, CAP)], sem)
            return carry

        lax.fori_loop(0, GRP, di, 0)

    def drain_task(cb, xb, sem):
        pltpu.make_async_copy(cells_hbm.at[pl.ds(0, GRP * CAP)], cb, sem).wait()
        pltpu.make_async_copy(xs_hbm.at[pl.ds(0, GRP * CAP)], xb, sem).wait()

    def process_task(t, cb, xb):
        b = b0 + lax.shift_right_logical(t, 2)
        g = t & 3

        def plist(j, carry):
            w = g * GRP + j
            n16 = plsc.load_gather(
                cnts_all, [jnp.broadcast_to(w * NBP + b, (16,)).astype(jnp.int32)])
            n = n16[0]
            base = j * CAP
            nvec = lax.shift_right_logical(n, 4)

            def do_vec(v):
                cell = cb[pl.ds(base + v * 16, 16)]
                xv = xb[pl.ds(base + v * 16, 16)]
                cnt, last = plsc.scan_count(cell)
                plsc.store_scatter(grid_v, [cell], xv, mask=last)

            def vec2(v, c2):
                do_vec(2 * v)
                do_vec(2 * v + 1)
                return c2

            lax.fori_loop(0, lax.shift_right_logical(nvec, 1), vec2, 0)

            @pl.when((nvec & 1) == 1)
            def _():
                do_vec(nvec - 1)

            return carry

        lax.fori_loop(0, GRP, plist, 0)

    def sweep_bucket(t):
        b = b0 + lax.shift_right_logical(t, 2)

        def jloop(j, carry):
            def hred(hh, accs):
                res = []
                for u in range(4):
                    a = pl.ds((4 * hh + u) * GW + j * 16, 16)
                    g = grid_v[a]
                    grid_v[a] = neg16
                    res.append(jnp.maximum(accs[u], g))
                return tuple(res)

            accs = lax.fori_loop(0, H_GRID // 4, hred, (neg16,) * 4)
            m01 = jnp.maximum(accs[0], accs[1])
            m23 = jnp.maximum(accs[2], accs[3])
            col[pl.ds(j * 16, 16)] = jnp.maximum(m01, m23)
            return carry

        lax.fori_loop(0, GW // 16, jloop, 0)

        @pl.when(b < NB - 1)
        def _():
            pltpu.sync_copy(col, out_hbm.at[pl.ds(b * GW, GW)])

        @pl.when(b == NB - 1)
        def _():
            pltpu.sync_copy(col.at[pl.ds(0, W_GRID - (NB - 1) * GW)],
                            out_hbm.at[pl.ds((NB - 1) * GW,
                                             W_GRID - (NB - 1) * GW)])

    issue_task(jnp.int32(0), cbufA, xbufA, semA)

    def tp_body(tp, carry):
        t0 = 2 * tp
        t1 = 2 * tp + 1
        issue_task(t1, cbufB, xbufB, semB)
        drain_task(cbufA, xbufA, semA)
        process_task(t0, cbufA, xbufA)

        @pl.when(t1 + 1 < ntask)
        def _():
            issue_task(t1 + 1, cbufA, xbufA, semA)

        drain_task(cbufB, xbufB, semB)
        process_task(t1, cbufB, xbufB)

        @pl.when((t1 & 3) == 3)
        def _():
            sweep_bucket(t1)

        return carry

    lax.fori_loop(0, lax.shift_right_logical(ntask, 1), tp_body, 0)


def kernel(input_1, tensor_index, W1, b1, W2, b2, W3, b3, W4, b4):
    x = _mlp(input_1, W1, b1, W2, b2, W3, b3, W4, b4)
    h_col = tensor_index[:, 0].astype(jnp.int32)
    w_col = tensor_index[:, 1].astype(jnp.int32)

    mesh = plsc.VectorSubcoreMesh(core_axis_name="c", subcore_axis_name="s")
    cp = pltpu.CompilerParams()
    if "needs_layout_passes" in pltpu.CompilerParams.__dataclass_fields__:
        cp = dataclasses.replace(cp, needs_layout_passes=False)

    route = pl.kernel(
        _route_kernel,
        mesh=mesh,
        compiler_params=cp,
        out_type=[
            jax.ShapeDtypeStruct((NW * NB * CAP,), jnp.int32),
            jax.ShapeDtypeStruct((NW * NB * CAP,), jnp.float32),
            jax.ShapeDtypeStruct((NW * NBP,), jnp.int32),
        ],
        scratch_types=[
            pltpu.VMEM((NB * CAP,), jnp.int32),
            pltpu.VMEM((NB * CAP,), jnp.float32),
            pltpu.VMEM((NBP,), jnp.int32),
            pltpu.VMEM((CHUNK_P,), jnp.int32),
            pltpu.VMEM((CHUNK_P,), jnp.int32),
            pltpu.VMEM((CHUNK_P,), jnp.float32),
            pltpu.VMEM((CHUNK_P,), jnp.int32),
            pltpu.VMEM((CHUNK_P,), jnp.int32),
            pltpu.VMEM((CHUNK_P,), jnp.float32),
            pltpu.SemaphoreType.DMA,
            pltpu.SemaphoreType.DMA,
        ],
    )
    cells, xs, counts = route(h_col, w_col, x)

    reduce_k = pl.kernel(
        _reduce_kernel,
        mesh=mesh,
        compiler_params=cp,
        out_type=jax.ShapeDtypeStruct((W_GRID,), jnp.float32),
        scratch_types=[
            pltpu.VMEM((GRID_TOT,), jnp.float32),
            pltpu.VMEM((GRP * CAP,), jnp.int32),
            pltpu.VMEM((GRP * CAP,), jnp.float32),
            pltpu.VMEM((GRP * CAP,), jnp.int32),
            pltpu.VMEM((GRP * CAP,), jnp.float32),
            pltpu.VMEM((NW * NBP,), jnp.int32),
            pltpu.VMEM((GW,), jnp.float32),
            pltpu.SemaphoreType.DMA,
            pltpu.SemaphoreType.DMA,
        ],
    )
    out = reduce_k(cells, xs, counts).reshape(1, W_GRID, 1)
    return (out, 1)


# speedup vs baseline: 1.0461x; 1.0461x over previous
import dataclasses
import functools

import jax
import jax.numpy as jnp
from jax import lax
from jax.experimental import pallas as pl
from jax.experimental.pallas import tpu as pltpu
from jax.experimental.pallas import tpu_sc as plsc

K = 1000000
H_GRID = 200
W_GRID = 70400

NW = 32
BSH = 9
GW = 512
NB = (W_GRID + GW - 1) // GW
NBP = 144
GRID_CELLS = H_GRID * GW
DUMP = GRID_CELLS
GRID_TOT = GRID_CELLS + 16
CAP = 432
NEG = -9999999.0

NVEC = K // 16
VPW = NVEC // NW
VEXTRA = NVEC - VPW * NW
CHUNK_V = 64
CHUNK_P = CHUNK_V * 16
NCHUNK = (VPW + 1 + CHUNK_V - 1) // CHUNK_V

GRP = 8
NGRP = NW // GRP

BPW = NB // NW
WEX = NB - BPW * NW


def _mlp_body(h_ref, w1_ref, b1_ref, w2_ref, b2_ref, w3_ref, b3_ref,
              w4_ref, b4_ref, o_ref):
    a = h_ref[...].reshape(4, h_ref.shape[3])
    z = jnp.maximum(jnp.dot(w1_ref[...], a,
                            preferred_element_type=jnp.float32) + b1_ref[...], 0.0)
    z = jnp.maximum(jnp.dot(w2_ref[...], z,
                            preferred_element_type=jnp.float32) + b2_ref[...], 0.0)
    z = jnp.maximum(jnp.dot(w3_ref[...], z,
                            preferred_element_type=jnp.float32) + b3_ref[...], 0.0)
    o_ref[...] = (jnp.dot(w4_ref[...], z,
                          preferred_element_type=jnp.float32)
                  + b4_ref[...]).reshape(o_ref.shape[0])


def _mlp(h2d, W1, b1, W2, b2, W3, b3, W4, b4):
    BK = 65536
    grid = (pl.cdiv(K, BK),)
    full = lambda shp: pl.BlockSpec(shp, lambda i: (0, 0))
    return pl.pallas_call(
        _mlp_body,
        grid=grid,
        in_specs=[
            pl.BlockSpec((1, 4, 1, BK), lambda i: (0, 0, 0, i)),
            full((18, 4)), full((18, 1)),
            full((36, 18)), full((36, 1)),
            full((36, 36)), full((36, 1)),
            full((1, 36)), full((1, 1)),
        ],
        out_specs=pl.BlockSpec((BK,), lambda i: (i,)),
        out_shape=jax.ShapeDtypeStruct((K,), jnp.float32),
    )(h2d, W1, b1[:, None], W2, b2[:, None], W3, b3[:, None], W4, b4[:, None])


def _route_kernel(h_hbm, w_hbm, x_hbm, cells_hbm, xs_hbm, counts_hbm,
                  stage_c, stage_x, counts,
                  hbuf0, wbuf0, xbuf0, hbuf1, wbuf1, xbuf1, sem0, sem1):
    wid = lax.axis_index("s") * 2 + lax.axis_index("c")
    v0 = wid * VPW + jnp.minimum(wid, VEXTRA)
    nv = VPW + jnp.where(wid < VEXTRA, 1, 0)

    zero16 = jnp.zeros((16,), jnp.int32)
    for j in range(0, NBP, 16):
        counts[pl.ds(j, 16)] = zero16

    iota = lax.iota(jnp.int32, 16)

    bufs = [(hbuf0, wbuf0, xbuf0), (hbuf1, wbuf1, xbuf1)]
    sems = [sem0, sem1]

    def issue(c, bi):
        off = (v0 + c * CHUNK_V) * 16
        offc = jnp.minimum(off, K - CHUNK_P)
        return [
            pltpu.async_copy(h_hbm.at[pl.ds(offc, CHUNK_P)],
                             bufs[bi][0], sems[bi]),
            pltpu.async_copy(w_hbm.at[pl.ds(offc, CHUNK_P)],
                             bufs[bi][1], sems[bi]),
            pltpu.async_copy(x_hbm.at[pl.ds(offc, CHUNK_P)],
                             bufs[bi][2], sems[bi]),
        ]

    hs = issue(0, 0)
    for c in range(NCHUNK):
        hs_next = issue(c + 1, (c + 1) % 2) if c + 1 < NCHUNK else None
        for hdl in hs:
            hdl.wait()
        hs = hs_next
        hb, wb, xb = bufs[c % 2]
        off = (v0 + c * CHUNK_V) * 16
        base = off - jnp.minimum(off, K - CHUNK_P)
        nproc = jnp.clip(nv - c * CHUNK_V, 0, CHUNK_V)

        def body(v, carry, hb=hb, wb=wb, xb=xb, base=base):
            p = base + v * 16
            hh = hb[pl.ds(p, 16)]
            ww = wb[pl.ds(p, 16)]
            xx = xb[pl.ds(p, 16)]
            b = lax.shift_right_logical(ww, BSH)
            wl = ww & (GW - 1)
            cell = hh * GW + wl
            cnt, last = plsc.scan_count(b)
            basec = plsc.load_gather(counts, [b])
            slot = basec + cnt - 1
            addr = b * CAP + slot
            plsc.store_scatter(stage_c, [addr], cell)
            plsc.store_scatter(stage_x, [addr], xx)
            plsc.store_scatter(counts, [b], basec + cnt, mask=last)
            return carry

        lax.fori_loop(0, nproc, body, 0)

    dump16 = jnp.full((16,), DUMP, jnp.int32)
    neg16 = jnp.full((16,), NEG, jnp.float32)

    for g in range(NBP // 16):
        n16 = counts[pl.ds(g * 16, 16)]
        pad16 = (-n16) & 15
        bvec = g * 16 + iota
        base_addr = bvec * CAP + n16
        for j in range(15):
            mask = (j < pad16) & (bvec < NB)
            plsc.store_scatter(stage_c, [base_addr + j], dump16, mask=mask)
            plsc.store_scatter(stage_x, [base_addr + j], neg16, mask=mask)
        counts[pl.ds(g * 16, 16)] = n16 + pad16

    pltpu.sync_copy(stage_c, cells_hbm.at[pl.ds(wid * (NB * CAP), NB * CAP)])
    pltpu.sync_copy(stage_x, xs_hbm.at[pl.ds(wid * (NB * CAP), NB * CAP)])
    pltpu.sync_copy(counts, counts_hbm.at[pl.ds(wid * NBP, NBP)])


def _reduce_kernel(cells_hbm, xs_hbm, counts_hbm, out_hbm,
                   grid_v, cbufA, xbufA, cbufB, xbufB, cnts_all, col,
                   semA, semB):
    wid = lax.axis_index("s") * 2 + lax.axis_index("c")
    neg16 = jnp.full((16,), NEG, jnp.float32)
    b0 = wid * BPW + jnp.minimum(wid, WEX)
    nb = BPW + jnp.where(wid < WEX, 1, 0)
    ntask = nb * NGRP

    def ig(j, carry):
        for u in range(16):
            grid_v[pl.ds(j * 256 + u * 16, 16)] = neg16
        return carry

    lax.fori_loop(0, GRID_CELLS // 256, ig, 0)
    grid_v[pl.ds(GRID_CELLS, 16)] = neg16

    pltpu.sync_copy(counts_hbm, cnts_all)

    def issue_task(t, cb, xb, sem):
        b = b0 + lax.shift_right_logical(t, 2)
        g = t & 3

        def di(j, carry):
            w = g * GRP + j
            src = (w * NB + b) * CAP
            pltpu.async_copy(cells_hbm.at[pl.ds(src, CAP)],
                             cb.at[pl.ds(j * CAP, CAP)], sem)
            pltpu.async_copy(xs_hbm.at[pl.ds(src, CAP)],
                             xb.at[pl.ds(j * CAP, CAP)], sem)
            return carry

        lax.fori_loop(0, GRP, di, 0)

    def drain_task(cb, xb, sem):
        pltpu.make_async_copy(cells_hbm.at[pl.ds(0, GRP * CAP)], cb, sem).wait()
        pltpu.make_async_copy(xs_hbm.at[pl.ds(0, GRP * CAP)], xb, sem).wait()

    def process_task(t, cb, xb):
        b = b0 + lax.shift_right_logical(t, 2)
        g = t & 3

        def plist(j, carry):
            w = g * GRP + j
            n16 = plsc.load_gather(
                cnts_all, [jnp.broadcast_to(w * NBP + b, (16,)).astype(jnp.int32)])
            n = n16[0]
            base = j * CAP

            def vec(v, c2):
                cell = cb[pl.ds(base + v * 16, 16)]
                xv = xb[pl.ds(base + v * 16, 16)]
                cnt, last = plsc.scan_count(cell)
                plsc.store_scatter(grid_v, [cell], xv, mask=last)
                return c2

            lax.fori_loop(0, lax.shift_right_logical(n, 4), vec, 0)
            return carry

        lax.fori_loop(0, GRP, plist, 0)

    def sweep_bucket(t):
        b = b0 + lax.shift_right_logical(t, 2)

        def jloop(j, carry):
            def hred(hh, accs):
                res = []
                for u in range(8):
                    a = pl.ds((8 * hh + u) * GW + j * 16, 16)
                    g = grid_v[a]
                    grid_v[a] = neg16
                    res.append(jnp.maximum(accs[u], g))
                return tuple(res)

            accs = lax.fori_loop(0, H_GRID // 8, hred, (neg16,) * 8)
            m = list(accs)
            while len(m) > 1:
                m = [jnp.maximum(m[2 * i], m[2 * i + 1])
                     for i in range(len(m) // 2)]
            col[pl.ds(j * 16, 16)] = m[0]
            return carry

        lax.fori_loop(0, GW // 16, jloop, 0)

        @pl.when(b < NB - 1)
        def _():
            pltpu.sync_copy(col, out_hbm.at[pl.ds(b * GW, GW)])

        @pl.when(b == NB - 1)
        def _():
            pltpu.sync_copy(col.at[pl.ds(0, W_GRID - (NB - 1) * GW)],
                            out_hbm.at[pl.ds((NB - 1) * GW,
                                             W_GRID - (NB - 1) * GW)])

    issue_task(jnp.int32(0), cbufA, xbufA, semA)

    def tp_body(tp, carry):
        t0 = 2 * tp
        t1 = 2 * tp + 1
        issue_task(t1, cbufB, xbufB, semB)
        drain_task(cbufA, xbufA, semA)
        process_task(t0, cbufA, xbufA)

        @pl.when(t1 + 1 < ntask)
        def _():
            issue_task(t1 + 1, cbufA, xbufA, semA)

        drain_task(cbufB, xbufB, semB)
        process_task(t1, cbufB, xbufB)

        @pl.when((t1 & 3) == 3)
        def _():
            sweep_bucket(t1)

        return carry

    lax.fori_loop(0, lax.shift_right_logical(ntask, 1), tp_body, 0)


def kernel(input_1, tensor_index, W1, b1, W2, b2, W3, b3, W4, b4):
    x = _mlp(input_1, W1, b1, W2, b2, W3, b3, W4, b4)
    h_col = tensor_index[:, 0].astype(jnp.int32)
    w_col = tensor_index[:, 1].astype(jnp.int32)

    mesh = plsc.VectorSubcoreMesh(core_axis_name="c", subcore_axis_name="s")
    cp = pltpu.CompilerParams()
    if "needs_layout_passes" in pltpu.CompilerParams.__dataclass_fields__:
        cp = dataclasses.replace(cp, needs_layout_passes=False)

    route = pl.kernel(
        _route_kernel,
        mesh=mesh,
        compiler_params=cp,
        out_type=[
            jax.ShapeDtypeStruct((NW * NB * CAP,), jnp.int32),
            jax.ShapeDtypeStruct((NW * NB * CAP,), jnp.float32),
            jax.ShapeDtypeStruct((NW * NBP,), jnp.int32),
        ],
        scratch_types=[
            pltpu.VMEM((NB * CAP,), jnp.int32),
            pltpu.VMEM((NB * CAP,), jnp.float32),
            pltpu.VMEM((NBP,), jnp.int32),
            pltpu.VMEM((CHUNK_P,), jnp.int32),
            pltpu.VMEM((CHUNK_P,), jnp.int32),
            pltpu.VMEM((CHUNK_P,), jnp.float32),
            pltpu.VMEM((CHUNK_P,), jnp.int32),
            pltpu.VMEM((CHUNK_P,), jnp.int32),
            pltpu.VMEM((CHUNK_P,), jnp.float32),
            pltpu.SemaphoreType.DMA,
            pltpu.SemaphoreType.DMA,
        ],
    )
    cells, xs, counts = route(h_col, w_col, x)

    reduce_k = pl.kernel(
        _reduce_kernel,
        mesh=mesh,
        compiler_params=cp,
        out_type=jax.ShapeDtypeStruct((W_GRID,), jnp.float32),
        scratch_types=[
            pltpu.VMEM((GRID_TOT,), jnp.float32),
            pltpu.VMEM((GRP * CAP,), jnp.int32),
            pltpu.VMEM((GRP * CAP,), jnp.float32),
            pltpu.VMEM((GRP * CAP,), jnp.int32),
            pltpu.VMEM((GRP * CAP,), jnp.float32),
            pltpu.VMEM((NW * NBP,), jnp.int32),
            pltpu.VMEM((GW,), jnp.float32),
            pltpu.SemaphoreType.DMA,
            pltpu.SemaphoreType.DMA,
        ],
    )
    out = reduce_k(cells, xs, counts).reshape(1, W_GRID, 1)
    return (out, 1)
